# baseline (speedup 1.0000x reference)
import jax
import jax.numpy as jnp
from jax import lax
from jax.experimental import pallas as pl
from jax.experimental.pallas import tpu as pltpu

B, S, D = 1, 2048, 2048
NH, NKV = 16, 8
HD = D // NH
DFF = 5632
E, K = 8, 2
R = 16
SCALE = 32.0 / 16.0
EPS = 1e-5

TS_A = 256
TS_Q = 256
TS_C = 256
TS_E = 256
F_E = 512
NJ = DFF // F_E

BF = jnp.bfloat16
F32 = jnp.float32


def _dot(a, b):
    return jnp.dot(a, b, preferred_element_type=F32)


def _qkv_body(x_ref, nw_ref, wq_ref, wk_ref, wv_ref, cc_ref, ss_ref, p_ref,
              q_ref, k_ref, v_ref):
    x = x_ref[...]
    var = jnp.mean(x * x, axis=-1, keepdims=True)
    h = (x * lax.rsqrt(var + EPS) * nw_ref[...]).astype(BF)
    q = _dot(h, wq_ref[...])
    k = _dot(h, wk_ref[...])
    v_ref[...] = _dot(h, wv_ref[...]).astype(BF)
    cc = cc_ref[...]
    ss = ss_ref[...]
    p = p_ref[...]
    for hh in range(NH):
        qh = q[:, hh * HD:(hh + 1) * HD]
        sw = _dot(qh.astype(BF), p)
        q_ref[:, hh * HD:(hh + 1) * HD] = (qh * cc + sw * ss).astype(BF)
    for hh in range(NKV):
        kh = k[:, hh * HD:(hh + 1) * HD]
        sw = _dot(kh.astype(BF), p)
        k_ref[:, hh * HD:(hh + 1) * HD] = (kh * cc + sw * ss).astype(BF)


def _attn_body(q_ref, k_ref, v_ref, o_ref):
    i = pl.program_id(1)
    q = q_ref[...]
    k = k_ref[...]
    scores = lax.dot_general(q, k, (((1,), (1,)), ((), ())),
                             preferred_element_type=F32)
    scores = scores * (1.0 / (HD ** 0.5))
    row = lax.broadcasted_iota(jnp.int32, scores.shape, 0) + i * TS_Q
    col = lax.broadcasted_iota(jnp.int32, scores.shape, 1)
    scores = jnp.where(col <= row, scores, -1e9)
    m = jnp.max(scores, axis=-1, keepdims=True)
    p = jnp.exp(scores - m)
    att = (p / jnp.sum(p, axis=-1, keepdims=True)).astype(BF)
    o_ref[...] = lax.dot_general(att, v_ref[...], (((1,), (0,)), ((), ())),
                                 preferred_element_type=F32).astype(BF)


def _proj_router_body(attn_ref, data_ref, wo_ref, nw_ref, gw_ref,
                      la1_ref, la3_ref,
                      d2_ref, sn_ref, ew_ref, a1_ref, a3_ref):
    d2 = _dot(attn_ref[...], wo_ref[...]) + data_ref[...]
    d2_ref[...] = d2
    var = jnp.mean(d2 * d2, axis=-1, keepdims=True)
    sn = d2 * lax.rsqrt(var + EPS) * nw_ref[...]
    snb = sn.astype(BF)
    sn_ref[...] = snb
    a1_ref[...] = _dot(snb, la1_ref[...])
    a3_ref[...] = _dot(snb, la3_ref[...])
    logits = _dot(snb, gw_ref[...])
    mx = jnp.max(logits, axis=-1, keepdims=True)
    pexp = jnp.exp(logits - mx)
    prob = pexp / jnp.sum(pexp, axis=-1, keepdims=True)
    eidx = lax.broadcasted_iota(jnp.int32, prob.shape, 1)
    m1 = jnp.max(prob, axis=-1, keepdims=True)
    i1 = jnp.min(jnp.where(prob == m1, eidx, E), axis=-1, keepdims=True)
    oh1 = eidx == i1
    p2 = jnp.where(oh1, -jnp.inf, prob)
    m2 = jnp.max(p2, axis=-1, keepdims=True)
    i2 = jnp.min(jnp.where(p2 == m2, eidx, E), axis=-1, keepdims=True)
    oh2 = eidx == i2
    denom = m1 + m2
    ew_ref[...] = (jnp.where(oh1, m1, 0.0) + jnp.where(oh2, m2, 0.0)) / denom


def _moe_body(sn_ref, d2_ref, ew_ref, a1_ref, a3_ref,
              w1_ref, w3_ref, bd1_ref, bd3_ref,
              w2_ref, bd2_ref, w2lb_ref,
              o_ref, acc_ref, u_ref):
    j = pl.program_id(1)

    @pl.when(j == 0)
    def _init():
        acc_ref[...] = jnp.zeros_like(acc_ref)
        u_ref[...] = jnp.zeros_like(u_ref)

    sn = sn_ref[...]
    cw1 = _dot(sn, w1_ref[...])
    cw3 = _dot(sn, w3_ref[...])
    ew = ew_ref[...]
    d1 = _dot(a1_ref[...].astype(BF), bd1_ref[0])
    d3 = _dot(a3_ref[...].astype(BF), bd3_ref[0])
    zsum = jnp.zeros((TS_E, F_E), F32)
    zs = []
    for e in range(E):
        lw1 = cw1 + d1[:, e * F_E:(e + 1) * F_E]
        lw3 = cw3 + d3[:, e * F_E:(e + 1) * F_E]
        s = lw1 * lax.logistic(lw1) * lw3
        z = ew[:, e:e + 1] * s
        zsum = zsum + z
        zs.append(z.astype(BF))
    zcat = jnp.concatenate(zs, axis=1)
    u_ref[...] += _dot(zcat, bd2_ref[0])
    acc_ref[...] += _dot(zsum.astype(BF), w2_ref[...])

    @pl.when(j == NJ - 1)
    def _fin():
        lora2 = _dot(u_ref[...].astype(BF), w2lb_ref[...])
        o_ref[...] = acc_ref[...] + d2_ref[...] + lora2


def _run(data, rope_cos, rope_sin, wq, wk, wv, wo, w1, w2, w3,
         gate_w, attn_norm_w, ffn_norm_w, w1_la, w1_lb, w3_la, w3_lb,
         w2_la, w2_lb):
    x = data.reshape(S, D)

    cc = jnp.stack([rope_cos, rope_cos], axis=-1).reshape(S, HD)
    ss = jnp.stack([-rope_sin, rope_sin], axis=-1).reshape(S, HD)
    ii = jnp.arange(HD)
    pmat = (ii[:, None] == (ii[None, :] ^ 1)).astype(BF)

    q, k, v = pl.pallas_call(
        _qkv_body,
        grid=(S // TS_A,),
        in_specs=[
            pl.BlockSpec((TS_A, D), lambda i: (i, 0)),
            pl.BlockSpec((1, D), lambda i: (0, 0)),
            pl.BlockSpec((D, NH * HD), lambda i: (0, 0)),
            pl.BlockSpec((D, NKV * HD), lambda i: (0, 0)),
            pl.BlockSpec((D, NKV * HD), lambda i: (0, 0)),
            pl.BlockSpec((TS_A, HD), lambda i: (i, 0)),
            pl.BlockSpec((TS_A, HD), lambda i: (i, 0)),
            pl.BlockSpec((HD, HD), lambda i: (0, 0)),
        ],
        out_specs=[
            pl.BlockSpec((TS_A, NH * HD), lambda i: (i, 0)),
            pl.BlockSpec((TS_A, NKV * HD), lambda i: (i, 0)),
            pl.BlockSpec((TS_A, NKV * HD), lambda i: (i, 0)),
        ],
        out_shape=[
            jax.ShapeDtypeStruct((S, NH * HD), BF),
            jax.ShapeDtypeStruct((S, NKV * HD), BF),
            jax.ShapeDtypeStruct((S, NKV * HD), BF),
        ],
        compiler_params=pltpu.CompilerParams(
            dimension_semantics=("arbitrary",)),
    )(x, attn_norm_w.reshape(1, D), wq.astype(BF), wk.astype(BF),
      wv.astype(BF), cc, ss, pmat)

    attn = pl.pallas_call(
        _attn_body,
        grid=(NH, S // TS_Q),
        in_specs=[
            pl.BlockSpec((TS_Q, HD), lambda h, i: (i, h)),
            pl.BlockSpec((S, HD), lambda h, i: (0, h // 2)),
            pl.BlockSpec((S, HD), lambda h, i: (0, h // 2)),
        ],
        out_specs=pl.BlockSpec((TS_Q, HD), lambda h, i: (i, h)),
        out_shape=jax.ShapeDtypeStruct((S, NH * HD), BF),
        compiler_params=pltpu.CompilerParams(
            dimension_semantics=("arbitrary", "arbitrary")),
    )(q, k, v)

    la1 = w1_la.transpose(1, 0, 2).reshape(D, E * R).astype(BF)
    la3 = w3_la.transpose(1, 0, 2).reshape(D, E * R).astype(BF)

    d2, sn, ew, a1, a3 = pl.pallas_call(
        _proj_router_body,
        grid=(S // TS_C,),
        in_specs=[
            pl.BlockSpec((TS_C, NH * HD), lambda i: (i, 0)),
            pl.BlockSpec((TS_C, D), lambda i: (i, 0)),
            pl.BlockSpec((NH * HD, D), lambda i: (0, 0)),
            pl.BlockSpec((1, D), lambda i: (0, 0)),
            pl.BlockSpec((D, E), lambda i: (0, 0)),
            pl.BlockSpec((D, E * R), lambda i: (0, 0)),
            pl.BlockSpec((D, E * R), lambda i: (0, 0)),
        ],
        out_specs=[
            pl.BlockSpec((TS_C, D), lambda i: (i, 0)),
            pl.BlockSpec((TS_C, D), lambda i: (i, 0)),
            pl.BlockSpec((TS_C, E), lambda i: (i, 0)),
            pl.BlockSpec((TS_C, E * R), lambda i: (i, 0)),
            pl.BlockSpec((TS_C, E * R), lambda i: (i, 0)),
        ],
        out_shape=[
            jax.ShapeDtypeStruct((S, D), F32),
            jax.ShapeDtypeStruct((S, D), BF),
            jax.ShapeDtypeStruct((S, E), F32),
            jax.ShapeDtypeStruct((S, E * R), F32),
            jax.ShapeDtypeStruct((S, E * R), F32),
        ],
        compiler_params=pltpu.CompilerParams(
            dimension_semantics=("arbitrary",)),
    )(attn, x, wo.astype(BF), ffn_norm_w.reshape(1, D), gate_w.astype(BF),
      la1, la3)

    eyeE = jnp.eye(E, dtype=F32)
    bd1 = jnp.einsum('ab,arjf->jarbf', eyeE,
                     (w1_lb * SCALE).reshape(E, R, NJ, F_E)
                     ).reshape(NJ, E * R, E * F_E).astype(BF)
    bd3 = jnp.einsum('ab,arjf->jarbf', eyeE,
                     (w3_lb * SCALE).reshape(E, R, NJ, F_E)
                     ).reshape(NJ, E * R, E * F_E).astype(BF)
    bd2 = jnp.einsum('ab,ajfr->jafbr', eyeE,
                     w2_la.reshape(E, NJ, F_E, R)
                     ).reshape(NJ, E * F_E, E * R).astype(BF)
    w2lb_v = (w2_lb * SCALE).reshape(E * R, D).astype(BF)

    out = pl.pallas_call(
        _moe_body,
        grid=(S // TS_E, NJ),
        in_specs=[
            pl.BlockSpec((TS_E, D), lambda s, j: (s, 0)),
            pl.BlockSpec((TS_E, D), lambda s, j: (s, 0)),
            pl.BlockSpec((TS_E, E), lambda s, j: (s, 0)),
            pl.BlockSpec((TS_E, E * R), lambda s, j: (s, 0)),
            pl.BlockSpec((TS_E, E * R), lambda s, j: (s, 0)),
            pl.BlockSpec((D, F_E), lambda s, j: (0, j)),
            pl.BlockSpec((D, F_E), lambda s, j: (0, j)),
            pl.BlockSpec((1, E * R, E * F_E), lambda s, j: (j, 0, 0)),
            pl.BlockSpec((1, E * R, E * F_E), lambda s, j: (j, 0, 0)),
            pl.BlockSpec((F_E, D), lambda s, j: (j, 0)),
            pl.BlockSpec((1, E * F_E, E * R), lambda s, j: (j, 0, 0)),
            pl.BlockSpec((E * R, D), lambda s, j: (0, 0)),
        ],
        out_specs=pl.BlockSpec((TS_E, D), lambda s, j: (s, 0)),
        out_shape=jax.ShapeDtypeStruct((S, D), F32),
        scratch_shapes=[
            pltpu.VMEM((TS_E, D), F32),
            pltpu.VMEM((TS_E, E * R), F32),
        ],
        compiler_params=pltpu.CompilerParams(
            dimension_semantics=("arbitrary", "arbitrary")),
    )(sn, d2, ew, a1, a3, w1.astype(BF), w3.astype(BF),
      bd1, bd3, w2.astype(BF), bd2, w2lb_v)

    return out.reshape(B, S, D)


def kernel(data, mask, rope_cos, rope_sin, wq, wk, wv, wo, w1, w2, w3,
           gate_w, attn_norm_w, ffn_norm_w, w1_la, w1_lb, w3_la, w3_lb,
           w2_la, w2_lb):
    del mask
    return _run(data, rope_cos, rope_sin, wq, wk, wv, wo, w1, w2, w3,
                gate_w, attn_norm_w, ffn_norm_w, w1_la, w1_lb, w3_la,
                w3_lb, w2_la, w2_lb)


# speedup vs baseline: 1.0661x; 1.0661x over previous
import jax
import jax.numpy as jnp
from jax import lax
from jax.experimental import pallas as pl
from jax.experimental.pallas import tpu as pltpu

B, S, D = 1, 2048, 2048
NH, NKV = 16, 8
HD = D // NH
DFF = 5632
E, K = 8, 2
R = 16
SCALE = 32.0 / 16.0
EPS = 1e-5

TS_A = 256
TS_Q = 256
TS_C = 256
TS_E = 256
F_E = 512
NJ = DFF // F_E

BF = jnp.bfloat16
F32 = jnp.float32


def _dot(a, b):
    return jnp.dot(a, b, preferred_element_type=F32)


def _qkv_body(x_ref, nw_ref, wq_ref, wk_ref, wv_ref, cc_ref, ss_ref, p_ref,
              q_ref, k_ref, v_ref):
    x = x_ref[...]
    var = jnp.mean(x * x, axis=-1, keepdims=True)
    h = (x * lax.rsqrt(var + EPS) * nw_ref[...]).astype(BF)
    q = _dot(h, wq_ref[...])
    k = _dot(h, wk_ref[...])
    v_ref[...] = _dot(h, wv_ref[...]).astype(BF)
    cc = cc_ref[...]
    ss = ss_ref[...]
    p = p_ref[...]
    for hh in range(NH):
        qh = q[:, hh * HD:(hh + 1) * HD]
        sw = _dot(qh.astype(BF), p)
        q_ref[:, hh * HD:(hh + 1) * HD] = (qh * cc + sw * ss).astype(BF)
    for hh in range(NKV):
        kh = k[:, hh * HD:(hh + 1) * HD]
        sw = _dot(kh.astype(BF), p)
        k_ref[:, hh * HD:(hh + 1) * HD] = (kh * cc + sw * ss).astype(BF)


def _attn_body(q_ref, k_ref, v_ref, o_ref):
    i = pl.program_id(1)
    q = q_ref[...]
    k = k_ref[...]
    scores = lax.dot_general(q, k, (((1,), (1,)), ((), ())),
                             preferred_element_type=F32)
    scores = scores * (1.0 / (HD ** 0.5))
    row = lax.broadcasted_iota(jnp.int32, scores.shape, 0) + i * TS_Q
    col = lax.broadcasted_iota(jnp.int32, scores.shape, 1)
    scores = jnp.where(col <= row, scores, -1e9)
    m = jnp.max(scores, axis=-1, keepdims=True)
    p = jnp.exp(scores - m)
    att = (p / jnp.sum(p, axis=-1, keepdims=True)).astype(BF)
    o_ref[...] = lax.dot_general(att, v_ref[...], (((1,), (0,)), ((), ())),
                                 preferred_element_type=F32).astype(BF)


def _proj_router_body(attn_ref, data_ref, wo_ref, nw_ref, gw_ref,
                      la1_ref, la3_ref,
                      d2_ref, sn_ref, ew_ref, a1_ref, a3_ref):
    d2 = _dot(attn_ref[...], wo_ref[...]) + data_ref[...]
    d2_ref[...] = d2
    var = jnp.mean(d2 * d2, axis=-1, keepdims=True)
    sn = d2 * lax.rsqrt(var + EPS) * nw_ref[...]
    snb = sn.astype(BF)
    sn_ref[...] = snb
    a1_ref[...] = _dot(snb, la1_ref[...])
    a3_ref[...] = _dot(snb, la3_ref[...])
    logits = _dot(snb, gw_ref[...])
    mx = jnp.max(logits, axis=-1, keepdims=True)
    pexp = jnp.exp(logits - mx)
    prob = pexp / jnp.sum(pexp, axis=-1, keepdims=True)
    eidx = lax.broadcasted_iota(jnp.int32, prob.shape, 1)
    m1 = jnp.max(prob, axis=-1, keepdims=True)
    i1 = jnp.min(jnp.where(prob == m1, eidx, E), axis=-1, keepdims=True)
    oh1 = eidx == i1
    p2 = jnp.where(oh1, -jnp.inf, prob)
    m2 = jnp.max(p2, axis=-1, keepdims=True)
    i2 = jnp.min(jnp.where(p2 == m2, eidx, E), axis=-1, keepdims=True)
    oh2 = eidx == i2
    denom = m1 + m2
    ew_ref[...] = (jnp.where(oh1, m1, 0.0) + jnp.where(oh2, m2, 0.0)) / denom


def _moe_body(sn_ref, d2_ref, ew_ref, a1_ref, a3_ref,
              w1_ref, w3_ref, bd1_ref, bd3_ref,
              w2_ref, w2la_ref, w2lb_ref,
              o_ref, acc_ref, u_ref):
    j = pl.program_id(1)

    @pl.when(j == 0)
    def _init():
        acc_ref[...] = jnp.zeros_like(acc_ref)
        u_ref[...] = jnp.zeros_like(u_ref)

    sn = sn_ref[...]
    cw1 = _dot(sn, w1_ref[...])
    cw3 = _dot(sn, w3_ref[...])
    ew = ew_ref[...]
    d1 = _dot(a1_ref[...].astype(BF), bd1_ref[...])
    d3 = _dot(a3_ref[...].astype(BF), bd3_ref[...])
    zsum = jnp.zeros((TS_E, F_E), F32)
    dus = []
    for e in range(E):
        lw1 = cw1 + d1[:, e * F_E:(e + 1) * F_E]
        lw3 = cw3 + d3[:, e * F_E:(e + 1) * F_E]
        s = lw1 * lax.logistic(lw1) * lw3
        z = ew[:, e:e + 1] * s
        zsum = zsum + z
        dus.append(_dot(z.astype(BF), w2la_ref[e]))
    u_ref[...] += jnp.concatenate(dus, axis=1)
    acc_ref[...] += _dot(zsum.astype(BF), w2_ref[...])

    @pl.when(j == NJ - 1)
    def _fin():
        lora2 = _dot(u_ref[...].astype(BF), w2lb_ref[...])
        o_ref[...] = acc_ref[...] + d2_ref[...] + lora2


def _run(data, rope_cos, rope_sin, wq, wk, wv, wo, w1, w2, w3,
         gate_w, attn_norm_w, ffn_norm_w, w1_la, w1_lb, w3_la, w3_lb,
         w2_la, w2_lb):
    x = data.reshape(S, D)

    cc = jnp.stack([rope_cos, rope_cos], axis=-1).reshape(S, HD)
    ss = jnp.stack([-rope_sin, rope_sin], axis=-1).reshape(S, HD)
    ii = jnp.arange(HD)
    pmat = (ii[:, None] == (ii[None, :] ^ 1)).astype(BF)

    q, k, v = pl.pallas_call(
        _qkv_body,
        grid=(S // TS_A,),
        in_specs=[
            pl.BlockSpec((TS_A, D), lambda i: (i, 0)),
            pl.BlockSpec((1, D), lambda i: (0, 0)),
            pl.BlockSpec((D, NH * HD), lambda i: (0, 0)),
            pl.BlockSpec((D, NKV * HD), lambda i: (0, 0)),
            pl.BlockSpec((D, NKV * HD), lambda i: (0, 0)),
            pl.BlockSpec((TS_A, HD), lambda i: (i, 0)),
            pl.BlockSpec((TS_A, HD), lambda i: (i, 0)),
            pl.BlockSpec((HD, HD), lambda i: (0, 0)),
        ],
        out_specs=[
            pl.BlockSpec((TS_A, NH * HD), lambda i: (i, 0)),
            pl.BlockSpec((TS_A, NKV * HD), lambda i: (i, 0)),
            pl.BlockSpec((TS_A, NKV * HD), lambda i: (i, 0)),
        ],
        out_shape=[
            jax.ShapeDtypeStruct((S, NH * HD), BF),
            jax.ShapeDtypeStruct((S, NKV * HD), BF),
            jax.ShapeDtypeStruct((S, NKV * HD), BF),
        ],
        compiler_params=pltpu.CompilerParams(
            dimension_semantics=("arbitrary",)),
    )(x, attn_norm_w.reshape(1, D), wq.astype(BF), wk.astype(BF),
      wv.astype(BF), cc, ss, pmat)

    attn = pl.pallas_call(
        _attn_body,
        grid=(NH, S // TS_Q),
        in_specs=[
            pl.BlockSpec((TS_Q, HD), lambda h, i: (i, h)),
            pl.BlockSpec((S, HD), lambda h, i: (0, h // 2)),
            pl.BlockSpec((S, HD), lambda h, i: (0, h // 2)),
        ],
        out_specs=pl.BlockSpec((TS_Q, HD), lambda h, i: (i, h)),
        out_shape=jax.ShapeDtypeStruct((S, NH * HD), BF),
        compiler_params=pltpu.CompilerParams(
            dimension_semantics=("arbitrary", "arbitrary")),
    )(q, k, v)

    la1 = w1_la.transpose(1, 0, 2).reshape(D, E * R).astype(BF)
    la3 = w3_la.transpose(1, 0, 2).reshape(D, E * R).astype(BF)

    d2, sn, ew, a1, a3 = pl.pallas_call(
        _proj_router_body,
        grid=(S // TS_C,),
        in_specs=[
            pl.BlockSpec((TS_C, NH * HD), lambda i: (i, 0)),
            pl.BlockSpec((TS_C, D), lambda i: (i, 0)),
            pl.BlockSpec((NH * HD, D), lambda i: (0, 0)),
            pl.BlockSpec((1, D), lambda i: (0, 0)),
            pl.BlockSpec((D, E), lambda i: (0, 0)),
            pl.BlockSpec((D, E * R), lambda i: (0, 0)),
            pl.BlockSpec((D, E * R), lambda i: (0, 0)),
        ],
        out_specs=[
            pl.BlockSpec((TS_C, D), lambda i: (i, 0)),
            pl.BlockSpec((TS_C, D), lambda i: (i, 0)),
            pl.BlockSpec((TS_C, E), lambda i: (i, 0)),
            pl.BlockSpec((TS_C, E * R), lambda i: (i, 0)),
            pl.BlockSpec((TS_C, E * R), lambda i: (i, 0)),
        ],
        out_shape=[
            jax.ShapeDtypeStruct((S, D), F32),
            jax.ShapeDtypeStruct((S, D), BF),
            jax.ShapeDtypeStruct((S, E), F32),
            jax.ShapeDtypeStruct((S, E * R), F32),
            jax.ShapeDtypeStruct((S, E * R), F32),
        ],
        compiler_params=pltpu.CompilerParams(
            dimension_semantics=("arbitrary",)),
    )(attn, x, wo.astype(BF), ffn_norm_w.reshape(1, D), gate_w.astype(BF),
      la1, la3)

    eyeE = jnp.eye(E, dtype=BF).reshape(E, 1, 1, E, 1)
    bd1 = (eyeE * (w1_lb * SCALE).astype(BF).reshape(E, R, NJ, 1, F_E)
           ).reshape(E * R, NJ * E * F_E)
    bd3 = (eyeE * (w3_lb * SCALE).astype(BF).reshape(E, R, NJ, 1, F_E)
           ).reshape(E * R, NJ * E * F_E)
    w2lb_v = (w2_lb * SCALE).reshape(E * R, D).astype(BF)

    out = pl.pallas_call(
        _moe_body,
        grid=(S // TS_E, NJ),
        in_specs=[
            pl.BlockSpec((TS_E, D), lambda s, j: (s, 0)),
            pl.BlockSpec((TS_E, D), lambda s, j: (s, 0)),
            pl.BlockSpec((TS_E, E), lambda s, j: (s, 0)),
            pl.BlockSpec((TS_E, E * R), lambda s, j: (s, 0)),
            pl.BlockSpec((TS_E, E * R), lambda s, j: (s, 0)),
            pl.BlockSpec((D, F_E), lambda s, j: (0, j)),
            pl.BlockSpec((D, F_E), lambda s, j: (0, j)),
            pl.BlockSpec((E * R, E * F_E), lambda s, j: (0, j)),
            pl.BlockSpec((E * R, E * F_E), lambda s, j: (0, j)),
            pl.BlockSpec((F_E, D), lambda s, j: (j, 0)),
            pl.BlockSpec((E, F_E, R), lambda s, j: (0, j, 0)),
            pl.BlockSpec((E * R, D), lambda s, j: (0, 0)),
        ],
        out_specs=pl.BlockSpec((TS_E, D), lambda s, j: (s, 0)),
        out_shape=jax.ShapeDtypeStruct((S, D), F32),
        scratch_shapes=[
            pltpu.VMEM((TS_E, D), F32),
            pltpu.VMEM((TS_E, E * R), F32),
        ],
        compiler_params=pltpu.CompilerParams(
            dimension_semantics=("arbitrary", "arbitrary")),
    )(sn, d2, ew, a1, a3, w1.astype(BF), w3.astype(BF),
      bd1, bd3, w2.astype(BF), w2_la.astype(BF), w2lb_v)

    return out.reshape(B, S, D)


def kernel(data, mask, rope_cos, rope_sin, wq, wk, wv, wo, w1, w2, w3,
           gate_w, attn_norm_w, ffn_norm_w, w1_la, w1_lb, w3_la, w3_lb,
           w2_la, w2_lb):
    del mask
    return _run(data, rope_cos, rope_sin, wq, wk, wv, wo, w1, w2, w3,
                gate_w, attn_norm_w, ffn_norm_w, w1_la, w1_lb, w3_la,
                w3_lb, w2_la, w2_lb)


# speedup vs baseline: 1.1466x; 1.0755x over previous
import jax
import jax.numpy as jnp
from jax import lax
from jax.experimental import pallas as pl
from jax.experimental.pallas import tpu as pltpu

B, S, D = 1, 2048, 2048
NH, NKV = 16, 8
HD = D // NH
DFF = 5632
E, K = 8, 2
R = 16
SCALE = 32.0 / 16.0
EPS = 1e-5

TS_A = 256
TS_Q = 256
TS_C = 256
TS_E = 256
F_E = 512
NJ = DFF // F_E

BF = jnp.bfloat16
F32 = jnp.float32


def _dot(a, b):
    return jnp.dot(a, b, preferred_element_type=F32)


def _qkv_body(x_ref, nw_ref, wq_ref, wk_ref, wv_ref, cc_ref, ss_ref, p_ref,
              q_ref, k_ref, v_ref):
    x = x_ref[...]
    var = jnp.mean(x * x, axis=-1, keepdims=True)
    h = (x * lax.rsqrt(var + EPS) * nw_ref[...]).astype(BF)
    q = _dot(h, wq_ref[...])
    k = _dot(h, wk_ref[...])
    v_ref[...] = _dot(h, wv_ref[...]).astype(BF)
    cc = cc_ref[...]
    ss = ss_ref[...]
    p = p_ref[...]
    for hh in range(NH):
        qh = q[:, hh * HD:(hh + 1) * HD]
        sw = _dot(qh.astype(BF), p)
        q_ref[:, hh * HD:(hh + 1) * HD] = (qh * cc + sw * ss).astype(BF)
    for hh in range(NKV):
        kh = k[:, hh * HD:(hh + 1) * HD]
        sw = _dot(kh.astype(BF), p)
        k_ref[:, hh * HD:(hh + 1) * HD] = (kh * cc + sw * ss).astype(BF)


def _attn_body(q_ref, k_ref, v_ref, o_ref):
    i = pl.program_id(1)
    q = q_ref[...]
    k = k_ref[...]
    scores = lax.dot_general(q, k, (((1,), (1,)), ((), ())),
                             preferred_element_type=F32)
    scores = scores * (1.0 / (HD ** 0.5))
    row = lax.broadcasted_iota(jnp.int32, scores.shape, 0) + i * TS_Q
    col = lax.broadcasted_iota(jnp.int32, scores.shape, 1)
    scores = jnp.where(col <= row, scores, -1e9)
    m = jnp.max(scores, axis=-1, keepdims=True)
    p = jnp.exp(scores - m)
    att = (p / jnp.sum(p, axis=-1, keepdims=True)).astype(BF)
    o_ref[...] = lax.dot_general(att, v_ref[...], (((1,), (0,)), ((), ())),
                                 preferred_element_type=F32).astype(BF)


def _proj_router_body(attn_ref, data_ref, wo_ref, nw_ref, gw_ref,
                      la1_ref, la3_ref,
                      d2_ref, sn_ref, ew_ref, a1_ref, a3_ref):
    d2 = _dot(attn_ref[...], wo_ref[...]) + data_ref[...]
    d2_ref[...] = d2
    var = jnp.mean(d2 * d2, axis=-1, keepdims=True)
    sn = d2 * lax.rsqrt(var + EPS) * nw_ref[...]
    snb = sn.astype(BF)
    sn_ref[...] = snb
    a1_ref[...] = _dot(snb, la1_ref[...])
    a3_ref[...] = _dot(snb, la3_ref[...])
    logits = _dot(snb, gw_ref[...])
    mx = jnp.max(logits, axis=-1, keepdims=True)
    pexp = jnp.exp(logits - mx)
    prob = pexp / jnp.sum(pexp, axis=-1, keepdims=True)
    eidx = lax.broadcasted_iota(jnp.int32, prob.shape, 1)
    m1 = jnp.max(prob, axis=-1, keepdims=True)
    i1 = jnp.min(jnp.where(prob == m1, eidx, E), axis=-1, keepdims=True)
    oh1 = eidx == i1
    p2 = jnp.where(oh1, -jnp.inf, prob)
    m2 = jnp.max(p2, axis=-1, keepdims=True)
    i2 = jnp.min(jnp.where(p2 == m2, eidx, E), axis=-1, keepdims=True)
    oh2 = eidx == i2
    denom = m1 + m2
    ew_ref[...] = (jnp.where(oh1, m1, 0.0) + jnp.where(oh2, m2, 0.0)) / denom


def _moe_body(sn_ref, d2_ref, ew_ref, a1_ref, a3_ref,
              w1_ref, w3_ref, bd1_ref, bd3_ref,
              w2_ref, w2la_ref, w2lb_ref,
              o_ref, acc_ref, u_ref):
    j = pl.program_id(1)

    @pl.when(j == 0)
    def _init():
        acc_ref[...] = jnp.zeros_like(acc_ref)
        u_ref[...] = jnp.zeros_like(u_ref)

    sn = sn_ref[...]
    cw1 = _dot(sn, w1_ref[...])
    cw3 = _dot(sn, w3_ref[...])
    ew = ew_ref[...]

    def bdiag(lb_ref):
        rows = []
        for e in range(E):
            blk = (lb_ref[e] * SCALE).astype(BF)
            pieces = []
            if e > 0:
                pieces.append(jnp.zeros((R, e * F_E), BF))
            pieces.append(blk)
            if e < E - 1:
                pieces.append(jnp.zeros((R, (E - 1 - e) * F_E), BF))
            rows.append(jnp.concatenate(pieces, axis=1) if len(pieces) > 1
                        else pieces[0])
        return jnp.concatenate(rows, axis=0)

    d1 = _dot(a1_ref[...].astype(BF), bdiag(bd1_ref))
    d3 = _dot(a3_ref[...].astype(BF), bdiag(bd3_ref))
    zsum = jnp.zeros((TS_E, F_E), F32)
    dus = []
    for e in range(E):
        lw1 = cw1 + d1[:, e * F_E:(e + 1) * F_E]
        lw3 = cw3 + d3[:, e * F_E:(e + 1) * F_E]
        s = lw1 * lax.logistic(lw1) * lw3
        z = ew[:, e:e + 1] * s
        zsum = zsum + z
        dus.append(_dot(z.astype(BF), w2la_ref[e]))
    u_ref[...] += jnp.concatenate(dus, axis=1)
    acc_ref[...] += _dot(zsum.astype(BF), w2_ref[...])

    @pl.when(j == NJ - 1)
    def _fin():
        lora2 = _dot(u_ref[...].astype(BF),
                     (w2lb_ref[...] * SCALE).astype(BF))
        o_ref[...] = acc_ref[...] + d2_ref[...] + lora2


def _run(data, rope_cos, rope_sin, wq, wk, wv, wo, w1, w2, w3,
         gate_w, attn_norm_w, ffn_norm_w, w1_la, w1_lb, w3_la, w3_lb,
         w2_la, w2_lb):
    x = data.reshape(S, D)

    cc = jnp.stack([rope_cos, rope_cos], axis=-1).reshape(S, HD)
    ss = jnp.stack([-rope_sin, rope_sin], axis=-1).reshape(S, HD)
    ii = jnp.arange(HD)
    pmat = (ii[:, None] == (ii[None, :] ^ 1)).astype(BF)

    q, k, v = pl.pallas_call(
        _qkv_body,
        grid=(S // TS_A,),
        in_specs=[
            pl.BlockSpec((TS_A, D), lambda i: (i, 0)),
            pl.BlockSpec((1, D), lambda i: (0, 0)),
            pl.BlockSpec((D, NH * HD), lambda i: (0, 0)),
            pl.BlockSpec((D, NKV * HD), lambda i: (0, 0)),
            pl.BlockSpec((D, NKV * HD), lambda i: (0, 0)),
            pl.BlockSpec((TS_A, HD), lambda i: (i, 0)),
            pl.BlockSpec((TS_A, HD), lambda i: (i, 0)),
            pl.BlockSpec((HD, HD), lambda i: (0, 0)),
        ],
        out_specs=[
            pl.BlockSpec((TS_A, NH * HD), lambda i: (i, 0)),
            pl.BlockSpec((TS_A, NKV * HD), lambda i: (i, 0)),
            pl.BlockSpec((TS_A, NKV * HD), lambda i: (i, 0)),
        ],
        out_shape=[
            jax.ShapeDtypeStruct((S, NH * HD), BF),
            jax.ShapeDtypeStruct((S, NKV * HD), BF),
            jax.ShapeDtypeStruct((S, NKV * HD), BF),
        ],
        compiler_params=pltpu.CompilerParams(
            dimension_semantics=("arbitrary",)),
    )(x, attn_norm_w.reshape(1, D), wq.astype(BF), wk.astype(BF),
      wv.astype(BF), cc, ss, pmat)

    attn = pl.pallas_call(
        _attn_body,
        grid=(NH, S // TS_Q),
        in_specs=[
            pl.BlockSpec((TS_Q, HD), lambda h, i: (i, h)),
            pl.BlockSpec((S, HD), lambda h, i: (0, h // 2)),
            pl.BlockSpec((S, HD), lambda h, i: (0, h // 2)),
        ],
        out_specs=pl.BlockSpec((TS_Q, HD), lambda h, i: (i, h)),
        out_shape=jax.ShapeDtypeStruct((S, NH * HD), BF),
        compiler_params=pltpu.CompilerParams(
            dimension_semantics=("arbitrary", "arbitrary")),
    )(q, k, v)

    la1 = w1_la.transpose(1, 0, 2).reshape(D, E * R).astype(BF)
    la3 = w3_la.transpose(1, 0, 2).reshape(D, E * R).astype(BF)

    d2, sn, ew, a1, a3 = pl.pallas_call(
        _proj_router_body,
        grid=(S // TS_C,),
        in_specs=[
            pl.BlockSpec((TS_C, NH * HD), lambda i: (i, 0)),
            pl.BlockSpec((TS_C, D), lambda i: (i, 0)),
            pl.BlockSpec((NH * HD, D), lambda i: (0, 0)),
            pl.BlockSpec((1, D), lambda i: (0, 0)),
            pl.BlockSpec((D, E), lambda i: (0, 0)),
            pl.BlockSpec((D, E * R), lambda i: (0, 0)),
            pl.BlockSpec((D, E * R), lambda i: (0, 0)),
        ],
        out_specs=[
            pl.BlockSpec((TS_C, D), lambda i: (i, 0)),
            pl.BlockSpec((TS_C, D), lambda i: (i, 0)),
            pl.BlockSpec((TS_C, E), lambda i: (i, 0)),
            pl.BlockSpec((TS_C, E * R), lambda i: (i, 0)),
            pl.BlockSpec((TS_C, E * R), lambda i: (i, 0)),
        ],
        out_shape=[
            jax.ShapeDtypeStruct((S, D), F32),
            jax.ShapeDtypeStruct((S, D), BF),
            jax.ShapeDtypeStruct((S, E), F32),
            jax.ShapeDtypeStruct((S, E * R), F32),
            jax.ShapeDtypeStruct((S, E * R), F32),
        ],
        compiler_params=pltpu.CompilerParams(
            dimension_semantics=("arbitrary",)),
    )(attn, x, wo.astype(BF), ffn_norm_w.reshape(1, D), gate_w.astype(BF),
      la1, la3)

    out = pl.pallas_call(
        _moe_body,
        grid=(S // TS_E, NJ),
        in_specs=[
            pl.BlockSpec((TS_E, D), lambda s, j: (s, 0)),
            pl.BlockSpec((TS_E, D), lambda s, j: (s, 0)),
            pl.BlockSpec((TS_E, E), lambda s, j: (s, 0)),
            pl.BlockSpec((TS_E, E * R), lambda s, j: (s, 0)),
            pl.BlockSpec((TS_E, E * R), lambda s, j: (s, 0)),
            pl.BlockSpec((D, F_E), lambda s, j: (0, j)),
            pl.BlockSpec((D, F_E), lambda s, j: (0, j)),
            pl.BlockSpec((E, R, F_E), lambda s, j: (0, 0, j)),
            pl.BlockSpec((E, R, F_E), lambda s, j: (0, 0, j)),
            pl.BlockSpec((F_E, D), lambda s, j: (j, 0)),
            pl.BlockSpec((E, F_E, R), lambda s, j: (0, j, 0)),
            pl.BlockSpec((E * R, D), lambda s, j: (0, 0)),
        ],
        out_specs=pl.BlockSpec((TS_E, D), lambda s, j: (s, 0)),
        out_shape=jax.ShapeDtypeStruct((S, D), F32),
        scratch_shapes=[
            pltpu.VMEM((TS_E, D), F32),
            pltpu.VMEM((TS_E, E * R), F32),
        ],
        compiler_params=pltpu.CompilerParams(
            dimension_semantics=("arbitrary", "arbitrary")),
    )(sn, d2, ew, a1, a3, w1.astype(BF), w3.astype(BF),
      w1_lb, w3_lb, w2.astype(BF), w2_la.astype(BF),
      w2_lb.reshape(E * R, D))

    return out.reshape(B, S, D)


def kernel(data, mask, rope_cos, rope_sin, wq, wk, wv, wo, w1, w2, w3,
           gate_w, attn_norm_w, ffn_norm_w, w1_la, w1_lb, w3_la, w3_lb,
           w2_la, w2_lb):
    del mask
    return _run(data, rope_cos, rope_sin, wq, wk, wv, wo, w1, w2, w3,
                gate_w, attn_norm_w, ffn_norm_w, w1_la, w1_lb, w3_la,
                w3_lb, w2_la, w2_lb)
